# baseline (speedup 1.0000x reference)
import jax
import jax.numpy as jnp
from jax.experimental import pallas as pl
from jax.experimental.pallas import tpu as pltpu


_M = 16384
_K = 1024
_N = 1024
_CH = 512
_NBUF = 4
_NCHUNK = _M // _CH


def _mapper_kernel(a_hbm, w_ref, b_ref, o_hbm, abuf, obuf, insem, outsem):
    def in_copy(i):
        return pltpu.make_async_copy(
            a_hbm.at[pl.ds(i * _CH, _CH), :], abuf.at[i % _NBUF],
            insem.at[i % _NBUF])

    def out_copy(i):
        return pltpu.make_async_copy(
            obuf.at[i % _NBUF], o_hbm.at[pl.ds(i * _CH, _CH), :],
            outsem.at[i % _NBUF])

    for i in range(_NBUF):
        in_copy(i).start()
    w = w_ref[...]
    bias = b_ref[...]
    for i in range(_NCHUNK):
        in_copy(i).wait()
        if i >= _NBUF:
            out_copy(i - _NBUF).wait()
        acc = jnp.dot(abuf[i % _NBUF], w, preferred_element_type=jnp.float32)
        obuf[i % _NBUF] = jnp.maximum(acc + bias, 0.0)
        out_copy(i).start()
        if i + _NBUF < _NCHUNK:
            in_copy(i + _NBUF).start()
    for i in range(_NCHUNK - _NBUF, _NCHUNK):
        out_copy(i).wait()


def kernel(flat_values, cu_seqlens, W, b):
    del cu_seqlens
    b2 = b.reshape(1, _N)
    return pl.pallas_call(
        _mapper_kernel,
        in_specs=[
            pl.BlockSpec(memory_space=pl.ANY),
            pl.BlockSpec(memory_space=pltpu.VMEM),
            pl.BlockSpec(memory_space=pltpu.VMEM),
        ],
        out_specs=pl.BlockSpec(memory_space=pl.ANY),
        out_shape=jax.ShapeDtypeStruct((_M, _N), jnp.float32),
        scratch_shapes=[
            pltpu.VMEM((_NBUF, _CH, _K), jnp.float32),
            pltpu.VMEM((_NBUF, _CH, _N), jnp.float32),
            pltpu.SemaphoreType.DMA((_NBUF,)),
            pltpu.SemaphoreType.DMA((_NBUF,)),
        ],
    )(flat_values, W, b2)


# speedup vs baseline: 1.0216x; 1.0216x over previous
import jax
import jax.numpy as jnp
from jax.experimental import pallas as pl
from jax.experimental.pallas import tpu as pltpu


_M = 16384
_K = 1024
_N = 1024
_CH = 2048
_NBUF = 3
_NCHUNK = _M // _CH


def _mapper_kernel(a_hbm, w_ref, b_ref, o_hbm, abuf, obuf, insem, outsem):
    def in_copy(i):
        return pltpu.make_async_copy(
            a_hbm.at[pl.ds(i * _CH, _CH), :], abuf.at[i % _NBUF],
            insem.at[i % _NBUF])

    def out_copy(i):
        return pltpu.make_async_copy(
            obuf.at[i % _NBUF], o_hbm.at[pl.ds(i * _CH, _CH), :],
            outsem.at[i % _NBUF])

    for i in range(_NBUF):
        in_copy(i).start()
    w = w_ref[...]
    bias = b_ref[...]
    for i in range(_NCHUNK):
        in_copy(i).wait()
        if i >= _NBUF:
            out_copy(i - _NBUF).wait()
        acc = jnp.dot(abuf[i % _NBUF], w, preferred_element_type=jnp.float32)
        obuf[i % _NBUF] = jnp.maximum(acc + bias, 0.0)
        out_copy(i).start()
        if i + _NBUF < _NCHUNK:
            in_copy(i + _NBUF).start()
    for i in range(_NCHUNK - _NBUF, _NCHUNK):
        out_copy(i).wait()


def kernel(flat_values, cu_seqlens, W, b):
    del cu_seqlens
    b2 = b.reshape(1, _N)
    return pl.pallas_call(
        _mapper_kernel,
        in_specs=[
            pl.BlockSpec(memory_space=pl.ANY),
            pl.BlockSpec(memory_space=pltpu.VMEM),
            pl.BlockSpec(memory_space=pltpu.VMEM),
        ],
        out_specs=pl.BlockSpec(memory_space=pl.ANY),
        out_shape=jax.ShapeDtypeStruct((_M, _N), jnp.float32),
        compiler_params=pltpu.CompilerParams(
            vmem_limit_bytes=63 * 1024 * 1024,
        ),
        scratch_shapes=[
            pltpu.VMEM((_NBUF, _CH, _K), jnp.float32),
            pltpu.VMEM((_NBUF, _CH, _N), jnp.float32),
            pltpu.SemaphoreType.DMA((_NBUF,)),
            pltpu.SemaphoreType.DMA((_NBUF,)),
        ],
    )(flat_values, W, b2)


# speedup vs baseline: 1.0553x; 1.0330x over previous
import jax
import jax.numpy as jnp
from jax.experimental import pallas as pl
from jax.experimental.pallas import tpu as pltpu


_M = 16384
_K = 1024
_N = 1024
_CH = 1024
_NBUF = 4
_NCHUNK = _M // _CH


def _mapper_kernel(a_hbm, w_ref, b_ref, o_hbm, abuf, obuf, insem, outsem):
    def in_copy(i):
        return pltpu.make_async_copy(
            a_hbm.at[pl.ds(i * _CH, _CH), :], abuf.at[i % _NBUF],
            insem.at[i % _NBUF])

    def out_copy(i):
        return pltpu.make_async_copy(
            obuf.at[i % _NBUF], o_hbm.at[pl.ds(i * _CH, _CH), :],
            outsem.at[i % _NBUF])

    for i in range(_NBUF):
        in_copy(i).start()
    w = w_ref[...]
    bias = b_ref[...]
    for i in range(_NCHUNK):
        in_copy(i).wait()
        if i >= _NBUF:
            out_copy(i - _NBUF).wait()
        acc = jnp.dot(abuf[i % _NBUF], w, preferred_element_type=jnp.float32)
        obuf[i % _NBUF] = jnp.maximum(acc + bias, 0.0)
        out_copy(i).start()
        if i + _NBUF < _NCHUNK:
            in_copy(i + _NBUF).start()
    for i in range(_NCHUNK - _NBUF, _NCHUNK):
        out_copy(i).wait()


def kernel(flat_values, cu_seqlens, W, b):
    del cu_seqlens
    b2 = b.reshape(1, _N)
    return pl.pallas_call(
        _mapper_kernel,
        in_specs=[
            pl.BlockSpec(memory_space=pl.ANY),
            pl.BlockSpec(memory_space=pltpu.VMEM),
            pl.BlockSpec(memory_space=pltpu.VMEM),
        ],
        out_specs=pl.BlockSpec(memory_space=pl.ANY),
        out_shape=jax.ShapeDtypeStruct((_M, _N), jnp.float32),
        scratch_shapes=[
            pltpu.VMEM((_NBUF, _CH, _K), jnp.float32),
            pltpu.VMEM((_NBUF, _CH, _N), jnp.float32),
            pltpu.SemaphoreType.DMA((_NBUF,)),
            pltpu.SemaphoreType.DMA((_NBUF,)),
        ],
    )(flat_values, W, b2)
